# baseline (speedup 1.0000x reference)
import functools

import jax
import jax.numpy as jnp
from jax import lax
from jax.experimental import pallas as pl
from jax.experimental.pallas import tpu as pltpu
from jax.experimental.pallas import tpu_sc as plsc

N = 10000
E = 320000
D = 128
NPAD = 10240
NC = 1
NS = 16
NW = NC * NS
CH = 128
NBUF = 2
IB = 4
CPW = 160
TOTCH = NW * CPW
EPAD = TOTCH * CH
RPT = NPAD // NS

_mesh = plsc.VectorSubcoreMesh(core_axis_name="c", subcore_axis_name="s",
                               num_cores=NC)


@functools.partial(
    pl.kernel,
    out_type=(
        jax.ShapeDtypeStruct((NPAD,), jnp.float32),
        jax.ShapeDtypeStruct((NPAD,), jnp.float32),
    ),
    mesh=_mesh,
    scratch_types=[
        pltpu.VMEM((CPW, CH), jnp.int32),
        pltpu.VMEM((CPW, CH), jnp.int32),
        pltpu.VMEM((CH,), jnp.float32),
        pltpu.VMEM((RPT,), jnp.float32),
        pltpu.VMEM_SHARED((NPAD,), jnp.float32),
        pltpu.VMEM_SHARED((NPAD,), jnp.float32),
        pltpu.SemaphoreType.DMA,
        pltpu.SemaphoreType.DMA,
    ],
)
def _deg_kernel(src_hbm, dst_hbm, dego_hbm, degi_hbm,
                srctab, dsttab, ones_v, z_v, dego_sh, degi_sh, sem0, sem1):
    sid = lax.axis_index("s")
    base = sid * CPW

    pltpu.sync_copy(src_hbm.at[pl.ds(base, CPW)], srctab)
    pltpu.sync_copy(dst_hbm.at[pl.ds(base, CPW)], dsttab)

    for j in range(CH // 16):
        ones_v[pl.ds(j * 16, 16)] = jnp.full((16,), 1.0, jnp.float32)

    def zb(i, _):
        z_v[pl.ds(i * 16, 16)] = jnp.zeros((16,), jnp.float32)
        return ()
    lax.fori_loop(0, RPT // 16, zb, ())
    pltpu.sync_copy(z_v, dego_sh.at[pl.ds(sid * RPT, RPT)])
    pltpu.sync_copy(z_v, degi_sh.at[pl.ds(sid * RPT, RPT)])
    plsc.subcore_barrier()

    def body(i, _):
        co = pltpu.async_copy(ones_v, dego_sh.at[srctab.at[i]], sem0, add=True)
        ci = pltpu.async_copy(ones_v, degi_sh.at[dsttab.at[i]], sem1, add=True)
        co.wait()
        ci.wait()
        return ()
    lax.fori_loop(0, CPW, body, ())
    plsc.subcore_barrier()

    pltpu.sync_copy(dego_sh.at[pl.ds(sid * RPT, RPT)],
                    dego_hbm.at[pl.ds(sid * RPT, RPT)])
    pltpu.sync_copy(degi_sh.at[pl.ds(sid * RPT, RPT)],
                    degi_hbm.at[pl.ds(sid * RPT, RPT)])


@functools.partial(
    pl.kernel,
    out_type=jax.ShapeDtypeStruct((NPAD, D), jnp.float32),
    mesh=_mesh,
    scratch_types=[
        pltpu.VMEM((IB, CH), jnp.int32),
        pltpu.VMEM((IB, CH), jnp.int32),
        pltpu.VMEM((NBUF, CH, D), jnp.float32),
        pltpu.VMEM_SHARED((NPAD, D), jnp.float32),
        [pltpu.SemaphoreType.DMA] * IB,
        [pltpu.SemaphoreType.DMA] * IB,
        [pltpu.SemaphoreType.DMA] * NBUF,
    ],
)
def _agg_kernel(src_hbm, dst_hbm, hs_hbm, out_hbm,
                srcbuf, dstbuf, rows_v, acc_sh, ssem, dsem, gsem):
    sid = lax.axis_index("s")
    base = sid * CPW

    def zr(r, _):
        def zc(j, _):
            rows_v[0, r, pl.ds(j * 16, 16)] = jnp.zeros((16,), jnp.float32)
            return ()
        lax.fori_loop(0, D // 16, zc, ())
        return ()
    lax.fori_loop(0, CH, zr, ())

    def zcopy(i, _):
        pltpu.sync_copy(rows_v.at[0], acc_sh.at[pl.ds(sid * RPT + i * CH, CH)])
        return ()
    lax.fori_loop(0, RPT // CH, zcopy, ())
    plsc.subcore_barrier()

    for k in range(IB):
        pltpu.async_copy(src_hbm.at[base + k], srcbuf.at[k], ssem[k])
        pltpu.async_copy(dst_hbm.at[base + k], dstbuf.at[k], dsem[k])
    for b in range(NBUF):
        pltpu.make_async_copy(src_hbm.at[base + b], srcbuf.at[b],
                              ssem[b]).wait()
        pltpu.async_copy(hs_hbm.at[srcbuf.at[b]], rows_v.at[b], gsem[b])

    def group(g, _):
        for k in range(IB):
            i = g * IB + k
            b = k % NBUF
            pltpu.make_async_copy(hs_hbm.at[srcbuf.at[k]],
                                  rows_v.at[b], gsem[b]).wait()
            pltpu.make_async_copy(dst_hbm.at[base + i], dstbuf.at[k],
                                  dsem[k]).wait()
            pltpu.sync_copy(rows_v.at[b], acc_sh.at[dstbuf.at[k]], add=True)

            @pl.when(i + IB < CPW)
            def _():
                pltpu.async_copy(src_hbm.at[base + i + IB], srcbuf.at[k],
                                 ssem[k])
                pltpu.async_copy(dst_hbm.at[base + i + IB], dstbuf.at[k],
                                 dsem[k])

            kk = (k + NBUF) % IB
            @pl.when(i + NBUF < CPW)
            def _():
                pltpu.make_async_copy(src_hbm.at[base + i + NBUF],
                                      srcbuf.at[kk], ssem[kk]).wait()
                pltpu.async_copy(hs_hbm.at[srcbuf.at[kk]], rows_v.at[b],
                                 gsem[b])
        return ()
    lax.fori_loop(0, CPW // IB, group, ())
    plsc.subcore_barrier()

    pltpu.sync_copy(acc_sh.at[pl.ds(sid * RPT, RPT)],
                    out_hbm.at[pl.ds(sid * RPT, RPT)])


_RB = 256
_GRID = NPAD // _RB


def _prep_body(f_ref, do_ref, di_ref, hs_ref, nsrc_ref, ndst_ref):
    ns = lax.rsqrt(do_ref[...] + 1.0)
    nd = lax.rsqrt(di_ref[...] + 1.0)
    nsrc_ref[...] = ns
    ndst_ref[...] = nd
    hs_ref[...] = f_ref[...] * ns


def _layer1_body(m_ref, hs_ref, nd_ref, ns_ref, w_ref, b_ref, out_ref):
    m = (m_ref[...] + hs_ref[...]) * nd_ref[...]
    h = jnp.dot(m, w_ref[...], preferred_element_type=jnp.float32) + b_ref[...]
    out_ref[...] = jnp.maximum(h, 0.0) * ns_ref[...]


def _layer2_body(m_ref, hs_ref, nd_ref, w_ref, b_ref, wp_ref, bp_ref,
                 out_ref):
    m = (m_ref[...] + hs_ref[...]) * nd_ref[...]
    h = jnp.dot(m, w_ref[...], preferred_element_type=jnp.float32) + b_ref[...]
    h = jnp.maximum(h, 0.0)
    out_ref[...] = jnp.sum(h * wp_ref[...], axis=1, keepdims=True) + bp_ref[...]


def _row_spec():
    return pl.BlockSpec((_RB, D), lambda i: (i, 0))


def _col_spec():
    return pl.BlockSpec((_RB, 1), lambda i: (i, 0))


def _full_spec(shape):
    return pl.BlockSpec(shape, lambda i: (0, 0))


_prep_call = pl.pallas_call(
    _prep_body,
    grid=(_GRID,),
    in_specs=[_row_spec(), _col_spec(), _col_spec()],
    out_specs=[_row_spec(), _col_spec(), _col_spec()],
    out_shape=[
        jax.ShapeDtypeStruct((NPAD, D), jnp.float32),
        jax.ShapeDtypeStruct((NPAD, 1), jnp.float32),
        jax.ShapeDtypeStruct((NPAD, 1), jnp.float32),
    ],
)

_layer1_call = pl.pallas_call(
    _layer1_body,
    grid=(_GRID,),
    in_specs=[_row_spec(), _row_spec(), _col_spec(), _col_spec(),
              _full_spec((D, D)), _full_spec((1, D))],
    out_specs=_row_spec(),
    out_shape=jax.ShapeDtypeStruct((NPAD, D), jnp.float32),
)

_layer2_call = pl.pallas_call(
    _layer2_body,
    grid=(_GRID,),
    in_specs=[_row_spec(), _row_spec(), _col_spec(),
              _full_spec((D, D)), _full_spec((1, D)), _full_spec((1, D)),
              _full_spec((1, 1))],
    out_specs=_col_spec(),
    out_shape=jax.ShapeDtypeStruct((NPAD, 1), jnp.float32),
)


def kernel(features, edge_index, W1, b1, W2, b2, Wp, bp):
    src = edge_index[0].astype(jnp.int32)
    dst = edge_index[1].astype(jnp.int32)
    padv = jnp.full((EPAD - E,), N, dtype=jnp.int32)
    src2d = jnp.concatenate([src, padv]).reshape(TOTCH, CH)
    dst2d = jnp.concatenate([dst, padv]).reshape(TOTCH, CH)

    dego, degi = _deg_kernel(src2d, dst2d)
    do = dego.reshape(NPAD, 1)
    di = degi.reshape(NPAD, 1)

    fpad = jnp.pad(features, ((0, NPAD - N), (0, 0)))
    hs0, nsrc, ndst = _prep_call(fpad, do, di)

    m1 = _agg_kernel(src2d, dst2d, hs0)
    h1s = _layer1_call(m1, hs0, ndst, nsrc, W1, b1.reshape(1, D))
    m2 = _agg_kernel(src2d, dst2d, h1s)
    logits = _layer2_call(m2, h1s, ndst, W2, b2.reshape(1, D),
                          Wp.reshape(1, D), bp.reshape(1, 1))
    return logits[:N]


# speedup vs baseline: 1.2929x; 1.2929x over previous
import functools

import jax
import jax.numpy as jnp
from jax import lax
from jax.experimental import pallas as pl
from jax.experimental.pallas import tpu as pltpu
from jax.experimental.pallas import tpu_sc as plsc

N = 10000
E = 320000
D = 128
NPAD = 10240
NC = 2
NS = 16
CH = 128
NBUF = 2
IB = 4
CPW0 = 120
CPW1 = 40
SC1BASE = NS * CPW0
TOTCH = NS * (CPW0 + CPW1)
EPAD = TOTCH * CH
RPT = NPAD // NS

_mesh = plsc.VectorSubcoreMesh(core_axis_name="c", subcore_axis_name="s",
                               num_cores=NC)


def _my_range(cid, sid):
    base = jnp.where(cid == 0, sid * CPW0, SC1BASE + sid * CPW1)
    cnt = jnp.where(cid == 0, CPW0, CPW1)
    wbase = jnp.minimum(base, TOTCH - CPW0)
    return base, cnt, base - wbase, wbase


@functools.partial(
    pl.kernel,
    out_type=(
        jax.ShapeDtypeStruct((NC * NPAD,), jnp.float32),
        jax.ShapeDtypeStruct((NC * NPAD,), jnp.float32),
    ),
    mesh=_mesh,
    scratch_types=[
        pltpu.VMEM((CPW0, CH), jnp.int32),
        pltpu.VMEM((CPW0, CH), jnp.int32),
        pltpu.VMEM((CH,), jnp.float32),
        pltpu.VMEM((RPT,), jnp.float32),
        pltpu.VMEM_SHARED((NPAD,), jnp.float32),
        pltpu.VMEM_SHARED((NPAD,), jnp.float32),
        pltpu.SemaphoreType.DMA,
        pltpu.SemaphoreType.DMA,
    ],
)
def _deg_kernel(src_hbm, dst_hbm, dego_hbm, degi_hbm,
                srctab, dsttab, ones_v, z_v, dego_sh, degi_sh, sem0, sem1):
    cid = lax.axis_index("c")
    sid = lax.axis_index("s")
    base, cnt, sh, wbase = _my_range(cid, sid)

    pltpu.sync_copy(src_hbm.at[pl.ds(wbase, CPW0)], srctab)
    pltpu.sync_copy(dst_hbm.at[pl.ds(wbase, CPW0)], dsttab)

    for j in range(CH // 16):
        ones_v[pl.ds(j * 16, 16)] = jnp.full((16,), 1.0, jnp.float32)

    def zb(i, _):
        z_v[pl.ds(i * 16, 16)] = jnp.zeros((16,), jnp.float32)
        return ()
    lax.fori_loop(0, RPT // 16, zb, ())
    pltpu.sync_copy(z_v, dego_sh.at[pl.ds(sid * RPT, RPT)])
    pltpu.sync_copy(z_v, degi_sh.at[pl.ds(sid * RPT, RPT)])
    plsc.subcore_barrier()

    def body(i, _):
        co = pltpu.async_copy(ones_v, dego_sh.at[srctab.at[sh + i]], sem0,
                              add=True)
        ci = pltpu.async_copy(ones_v, degi_sh.at[dsttab.at[sh + i]], sem1,
                              add=True)
        co.wait()
        ci.wait()
        return ()
    lax.fori_loop(0, cnt, body, ())
    plsc.subcore_barrier()

    off = cid * NPAD + sid * RPT
    pltpu.sync_copy(dego_sh.at[pl.ds(sid * RPT, RPT)],
                    dego_hbm.at[pl.ds(off, RPT)])
    pltpu.sync_copy(degi_sh.at[pl.ds(sid * RPT, RPT)],
                    degi_hbm.at[pl.ds(off, RPT)])


@functools.partial(
    pl.kernel,
    out_type=jax.ShapeDtypeStruct((NC * NPAD, D), jnp.float32),
    mesh=_mesh,
    scratch_types=[
        pltpu.VMEM((CPW0, CH), jnp.int32),
        pltpu.VMEM((IB, CH), jnp.int32),
        pltpu.VMEM((NBUF, CH, D), jnp.float32),
        pltpu.VMEM_SHARED((NPAD, D), jnp.float32),
        [pltpu.SemaphoreType.DMA] * IB,
        [pltpu.SemaphoreType.DMA] * NBUF,
    ],
)
def _agg_kernel(src_hbm, dst_hbm, hs_hbm, out_hbm,
                srctab, dstbuf, rows_v, acc_sh, dsem, gsem):
    cid = lax.axis_index("c")
    sid = lax.axis_index("s")
    base, cnt, sh, wbase = _my_range(cid, sid)

    pltpu.sync_copy(src_hbm.at[pl.ds(wbase, CPW0)], srctab)

    def zr(r, _):
        def zc(j, _):
            rows_v[0, r, pl.ds(j * 16, 16)] = jnp.zeros((16,), jnp.float32)
            return ()
        lax.fori_loop(0, D // 16, zc, ())
        return ()
    lax.fori_loop(0, CH, zr, ())

    def zcopy(i, _):
        pltpu.sync_copy(rows_v.at[0], acc_sh.at[pl.ds(sid * RPT + i * CH, CH)])
        return ()
    lax.fori_loop(0, RPT // CH, zcopy, ())
    plsc.subcore_barrier()

    for k in range(IB):
        pltpu.async_copy(dst_hbm.at[base + k], dstbuf.at[k], dsem[k])
    for b in range(NBUF):
        pltpu.async_copy(hs_hbm.at[srctab.at[sh + b]], rows_v.at[b], gsem[b])

    def group(g, _):
        for k in range(IB):
            i = g * IB + k
            b = k % NBUF
            pltpu.make_async_copy(hs_hbm.at[srctab.at[sh + i]],
                                  rows_v.at[b], gsem[b]).wait()
            pltpu.make_async_copy(dst_hbm.at[base + i], dstbuf.at[k],
                                  dsem[k]).wait()
            pltpu.sync_copy(rows_v.at[b], acc_sh.at[dstbuf.at[k]], add=True)

            @pl.when(i + IB < cnt)
            def _():
                pltpu.async_copy(dst_hbm.at[base + i + IB], dstbuf.at[k],
                                 dsem[k])

            @pl.when(i + NBUF < cnt)
            def _():
                pltpu.async_copy(hs_hbm.at[srctab.at[sh + i + NBUF]],
                                 rows_v.at[b], gsem[b])
        return ()
    lax.fori_loop(0, cnt // IB, group, ())
    plsc.subcore_barrier()

    off = cid * NPAD + sid * RPT
    pltpu.sync_copy(acc_sh.at[pl.ds(sid * RPT, RPT)],
                    out_hbm.at[pl.ds(off, RPT)])


_RB = 256
_GRID = NPAD // _RB


def _prep_body(f_ref, do0_ref, do1_ref, di0_ref, di1_ref,
               hs_ref, nsrc_ref, ndst_ref):
    ns = lax.rsqrt(do0_ref[...] + do1_ref[...] + 1.0)
    nd = lax.rsqrt(di0_ref[...] + di1_ref[...] + 1.0)
    nsrc_ref[...] = ns
    ndst_ref[...] = nd
    hs_ref[...] = f_ref[...] * ns


def _layer1_body(m0_ref, m1_ref, hs_ref, nd_ref, ns_ref, w_ref, b_ref,
                 out_ref):
    m = (m0_ref[...] + m1_ref[...] + hs_ref[...]) * nd_ref[...]
    h = jnp.dot(m, w_ref[...], preferred_element_type=jnp.float32) + b_ref[...]
    out_ref[...] = jnp.maximum(h, 0.0) * ns_ref[...]


def _layer2_body(m0_ref, m1_ref, hs_ref, nd_ref, w_ref, b_ref, wp_ref, bp_ref,
                 out_ref):
    m = (m0_ref[...] + m1_ref[...] + hs_ref[...]) * nd_ref[...]
    h = jnp.dot(m, w_ref[...], preferred_element_type=jnp.float32) + b_ref[...]
    h = jnp.maximum(h, 0.0)
    out_ref[...] = jnp.sum(h * wp_ref[...], axis=1, keepdims=True) + bp_ref[...]


def _row_spec(half=0):
    if half:
        return pl.BlockSpec((_RB, D), lambda i: (NPAD // _RB + i, 0))
    return pl.BlockSpec((_RB, D), lambda i: (i, 0))


def _col_spec(half=0):
    if half:
        return pl.BlockSpec((_RB, 1), lambda i: (NPAD // _RB + i, 0))
    return pl.BlockSpec((_RB, 1), lambda i: (i, 0))


def _full_spec(shape):
    return pl.BlockSpec(shape, lambda i: (0, 0))


_prep_call = pl.pallas_call(
    _prep_body,
    grid=(_GRID,),
    in_specs=[_row_spec(), _col_spec(), _col_spec(1), _col_spec(),
              _col_spec(1)],
    out_specs=[_row_spec(), _col_spec(), _col_spec()],
    out_shape=[
        jax.ShapeDtypeStruct((NPAD, D), jnp.float32),
        jax.ShapeDtypeStruct((NPAD, 1), jnp.float32),
        jax.ShapeDtypeStruct((NPAD, 1), jnp.float32),
    ],
)

_layer1_call = pl.pallas_call(
    _layer1_body,
    grid=(_GRID,),
    in_specs=[_row_spec(), _row_spec(1), _row_spec(), _col_spec(),
              _col_spec(), _full_spec((D, D)), _full_spec((1, D))],
    out_specs=_row_spec(),
    out_shape=jax.ShapeDtypeStruct((NPAD, D), jnp.float32),
)

_layer2_call = pl.pallas_call(
    _layer2_body,
    grid=(_GRID,),
    in_specs=[_row_spec(), _row_spec(1), _row_spec(), _col_spec(),
              _full_spec((D, D)), _full_spec((1, D)), _full_spec((1, D)),
              _full_spec((1, 1))],
    out_specs=_col_spec(),
    out_shape=jax.ShapeDtypeStruct((NPAD, 1), jnp.float32),
)


def kernel(features, edge_index, W1, b1, W2, b2, Wp, bp):
    src = edge_index[0].astype(jnp.int32)
    dst = edge_index[1].astype(jnp.int32)
    padv = jnp.full((EPAD - E,), N, dtype=jnp.int32)
    src2d = jnp.concatenate([src, padv]).reshape(TOTCH, CH)
    dst2d = jnp.concatenate([dst, padv]).reshape(TOTCH, CH)

    dego_p, degi_p = _deg_kernel(src2d, dst2d)
    dego2 = dego_p.reshape(NC * NPAD, 1)
    degi2 = degi_p.reshape(NC * NPAD, 1)

    fpad = jnp.pad(features, ((0, NPAD - N), (0, 0)))
    hs0, nsrc, ndst = _prep_call(fpad, dego2, dego2, degi2, degi2)

    m1 = _agg_kernel(src2d, dst2d, hs0)
    h1s = _layer1_call(m1, m1, hs0, ndst, nsrc, W1, b1.reshape(1, D))
    m2 = _agg_kernel(src2d, dst2d, h1s)
    logits = _layer2_call(m2, m2, h1s, ndst, W2, b2.reshape(1, D),
                          Wp.reshape(1, D), bp.reshape(1, 1))
    return logits[:N]
